# initial kernel scaffold (unmeasured)
import jax
import jax.numpy as jnp
from jax import lax
from jax.experimental import pallas as pl
from jax.experimental.pallas import tpu as pltpu

N_ROWS = 8192
N_COLS = 2048
CHUNK = 1024
N_CHUNKS = N_ROWS // CHUNK
EPS = 1e-6


def kernel(partial, resid, gamma):
    gamma2 = gamma.reshape(1, N_COLS)

    def body(
        partial_ref,
        resid_ref,
        gamma_ref,
        out_ref,
        own_f32,
        own_bf16,
        recv_bf16,
        resid_v,
        out_v,
        load_sems,
        store_sem,
        send_sems,
        recv_sems,
    ):
        my_x = lax.axis_index("x")
        my_y = lax.axis_index("y")
        nbr = (my_x, 1 - my_y)

        barrier_sem = pltpu.get_barrier_semaphore()
        pl.semaphore_signal(
            barrier_sem, inc=1, device_id=nbr,
            device_id_type=pl.DeviceIdType.MESH,
        )
        pl.semaphore_wait(barrier_sem, 1)

        for i in range(N_CHUNKS):
            rows = pl.ds(i * CHUNK, CHUNK)
            ld_p = pltpu.make_async_copy(
                partial_ref.at[0, rows, :], own_f32, load_sems.at[0]
            )
            ld_r = pltpu.make_async_copy(
                resid_ref.at[rows, :], resid_v, load_sems.at[1]
            )
            ld_p.start()
            ld_r.start()
            ld_p.wait()
            ld_r.wait()

            own_bf16[...] = own_f32[...].astype(jnp.bfloat16)
            rdma = pltpu.make_async_remote_copy(
                src_ref=own_bf16,
                dst_ref=recv_bf16.at[i],
                send_sem=send_sems.at[i],
                recv_sem=recv_sems.at[i],
                device_id=nbr,
                device_id_type=pl.DeviceIdType.MESH,
            )
            rdma.start()
            rdma.wait()

            y = own_f32[...] + recv_bf16[i].astype(jnp.float32) + resid_v[...]
            ms = jnp.mean(y * y, axis=-1, keepdims=True)
            out_v[...] = y * lax.rsqrt(ms + EPS) * gamma_ref[...]

            st = pltpu.make_async_copy(out_v, out_ref.at[rows, :], store_sem)
            st.start()
            st.wait()

    return pl.pallas_call(
        body,
        out_shape=jax.ShapeDtypeStruct((N_ROWS, N_COLS), jnp.float32),
        in_specs=[
            pl.BlockSpec(memory_space=pltpu.ANY),
            pl.BlockSpec(memory_space=pltpu.ANY),
            pl.BlockSpec(memory_space=pltpu.VMEM),
        ],
        out_specs=pl.BlockSpec(memory_space=pltpu.ANY),
        scratch_shapes=[
            pltpu.VMEM((CHUNK, N_COLS), jnp.float32),
            pltpu.VMEM((CHUNK, N_COLS), jnp.bfloat16),
            pltpu.VMEM((N_CHUNKS, CHUNK, N_COLS), jnp.bfloat16),
            pltpu.VMEM((CHUNK, N_COLS), jnp.float32),
            pltpu.VMEM((CHUNK, N_COLS), jnp.float32),
            pltpu.SemaphoreType.DMA((2,)),
            pltpu.SemaphoreType.DMA,
            pltpu.SemaphoreType.DMA((N_CHUNKS,)),
            pltpu.SemaphoreType.DMA((N_CHUNKS,)),
        ],
        compiler_params=pltpu.CompilerParams(collective_id=0),
    )(partial, resid, gamma2)


# baseline (device time: 542109 ns/iter reference)
import jax
import jax.numpy as jnp
from jax import lax
from jax.experimental import pallas as pl
from jax.experimental.pallas import tpu as pltpu

N_ROWS = 8192
N_COLS = 2048
CHUNK = 512
N_CHUNKS = N_ROWS // CHUNK
EPS = 1e-6


def kernel(partial, resid, gamma):
    gamma2 = gamma.reshape(1, N_COLS)

    def body(
        partial_ref,
        resid_ref,
        gamma_ref,
        out_ref,
        own_f32,
        own_bf16,
        recv_bf16,
        resid_v,
        out_v,
        load_sems,
        store_sem,
        send_sems,
        recv_sems,
    ):
        my_x = lax.axis_index("x")
        my_y = lax.axis_index("y")
        nbr = (my_x, 1 - my_y)

        barrier_sem = pltpu.get_barrier_semaphore()
        pl.semaphore_signal(
            barrier_sem, inc=1, device_id=nbr,
            device_id_type=pl.DeviceIdType.MESH,
        )
        pl.semaphore_wait(barrier_sem, 1)

        for i in range(N_CHUNKS):
            slot = i % 2
            rows = pl.ds(i * CHUNK, CHUNK)
            ld_p = pltpu.make_async_copy(
                partial_ref.at[0, rows, :], own_f32, load_sems.at[0]
            )
            ld_r = pltpu.make_async_copy(
                resid_ref.at[rows, :], resid_v, load_sems.at[1]
            )
            ld_p.start()
            ld_r.start()
            ld_p.wait()
            ld_r.wait()

            own_bf16[...] = own_f32[...].astype(jnp.bfloat16)
            rdma = pltpu.make_async_remote_copy(
                src_ref=own_bf16,
                dst_ref=recv_bf16.at[slot],
                send_sem=send_sems.at[slot],
                recv_sem=recv_sems.at[slot],
                device_id=nbr,
                device_id_type=pl.DeviceIdType.MESH,
            )
            rdma.start()
            rdma.wait()

            y = own_f32[...] + recv_bf16[slot].astype(jnp.float32) + resid_v[...]
            ms = jnp.mean(y * y, axis=-1, keepdims=True)
            out_v[...] = y * lax.rsqrt(ms + EPS) * gamma_ref[...]

            st = pltpu.make_async_copy(out_v, out_ref.at[rows, :], store_sem)
            st.start()
            st.wait()

    return pl.pallas_call(
        body,
        out_shape=jax.ShapeDtypeStruct((N_ROWS, N_COLS), jnp.float32),
        in_specs=[
            pl.BlockSpec(memory_space=pl.ANY),
            pl.BlockSpec(memory_space=pl.ANY),
            pl.BlockSpec(memory_space=pltpu.VMEM),
        ],
        out_specs=pl.BlockSpec(memory_space=pl.ANY),
        scratch_shapes=[
            pltpu.VMEM((CHUNK, N_COLS), jnp.float32),
            pltpu.VMEM((CHUNK, N_COLS), jnp.bfloat16),
            pltpu.VMEM((2, CHUNK, N_COLS), jnp.bfloat16),
            pltpu.VMEM((CHUNK, N_COLS), jnp.float32),
            pltpu.VMEM((CHUNK, N_COLS), jnp.float32),
            pltpu.SemaphoreType.DMA((2,)),
            pltpu.SemaphoreType.DMA,
            pltpu.SemaphoreType.DMA((2,)),
            pltpu.SemaphoreType.DMA((2,)),
        ],
        compiler_params=pltpu.CompilerParams(collective_id=0),
    )(partial, resid, gamma2)


# device time: 272609 ns/iter; 1.9886x vs baseline; 1.9886x over previous
import jax
import jax.numpy as jnp
from jax import lax
from jax.experimental import pallas as pl
from jax.experimental.pallas import tpu as pltpu

N_ROWS = 8192
N_COLS = 2048
HALF = N_ROWS // 2
CHUNK = 512
N_CHUNKS = HALF // CHUNK
EPS = 1e-6


def kernel(partial, resid, gamma):
    gamma2 = gamma.reshape(1, N_COLS)

    def body(
        partial_ref,
        resid_ref,
        gamma_ref,
        out_ref,
        own_f32,
        resid_v,
        own_bf16,
        recv_y,
        out_v,
        out_bf16,
        recv_x,
        xrecv_f32,
        pload_sems,
        rload_sems,
        store_sems,
        xstore_sem,
        ysend_sems,
        yrecv_sems,
        xsend_sems,
        xrecv_sems,
    ):
        my_x = lax.axis_index("x")
        my_y = lax.axis_index("y")
        ynbr = (my_x, 1 - my_y)
        xnbr = (1 - my_x, my_y)
        base_own = my_x * HALF
        base_oth = (1 - my_x) * HALF

        barrier_sem = pltpu.get_barrier_semaphore()
        for nbr in (ynbr, xnbr):
            pl.semaphore_signal(
                barrier_sem, inc=1, device_id=nbr,
                device_id_type=pl.DeviceIdType.MESH,
            )
        pl.semaphore_wait(barrier_sem, 2)

        def start_loads(i):
            s = i % 2
            rows = pl.ds(base_own + i * CHUNK, CHUNK)
            ld_p = pltpu.make_async_copy(
                partial_ref.at[0, rows, :], own_f32.at[s], pload_sems.at[s]
            )
            ld_r = pltpu.make_async_copy(
                resid_ref.at[rows, :], resid_v.at[s], rload_sems.at[s]
            )
            ld_p.start()
            ld_r.start()
            return ld_p, ld_r

        def y_rdma(i):
            s = i % 2
            return pltpu.make_async_remote_copy(
                src_ref=own_bf16.at[s],
                dst_ref=recv_y.at[s],
                send_sem=ysend_sems.at[s],
                recv_sem=yrecv_sems.at[s],
                device_id=ynbr,
                device_id_type=pl.DeviceIdType.MESH,
            )

        def x_rdma(i):
            return pltpu.make_async_remote_copy(
                src_ref=out_bf16.at[i % 2],
                dst_ref=recv_x.at[i % 4],
                send_sem=xsend_sems.at[i % 2],
                recv_sem=xrecv_sems.at[i % 4],
                device_id=xnbr,
                device_id_type=pl.DeviceIdType.MESH,
            )

        loads = [None] * N_CHUNKS
        ydescs = [None] * N_CHUNKS
        xdescs = [None] * N_CHUNKS
        stores = [None] * N_CHUNKS
        xstores = [None] * N_CHUNKS

        def handle_xrecv(j):
            xdescs[j].wait_recv()
            if j >= 1:
                xstores[j - 1].wait()
            xrecv_f32[...] = recv_x[j % 4].astype(jnp.float32)
            st = pltpu.make_async_copy(
                xrecv_f32,
                out_ref.at[pl.ds(base_oth + j * CHUNK, CHUNK), :],
                xstore_sem,
            )
            st.start()
            xstores[j] = st

        loads[0] = start_loads(0)
        loads[1] = start_loads(1)
        loads[0][0].wait()
        own_bf16[0] = own_f32[0].astype(jnp.bfloat16)
        ydescs[0] = y_rdma(0)
        ydescs[0].start()

        for i in range(N_CHUNKS):
            s = i % 2
            ydescs[i].wait_recv()

            if i + 1 < N_CHUNKS:
                loads[i + 1][0].wait()
                if i >= 1:
                    ydescs[i - 1].wait_send()
                own_bf16[(i + 1) % 2] = own_f32[(i + 1) % 2].astype(jnp.bfloat16)
                ydescs[i + 1] = y_rdma(i + 1)
                ydescs[i + 1].start()

            loads[i][1].wait()
            if i >= 2:
                stores[i - 2].wait()
            y = own_f32[s] + recv_y[s].astype(jnp.float32) + resid_v[s]
            ms = jnp.mean(y * y, axis=-1, keepdims=True)
            out_v[s] = y * lax.rsqrt(ms + EPS) * gamma_ref[...]

            st = pltpu.make_async_copy(
                out_v.at[s],
                out_ref.at[pl.ds(base_own + i * CHUNK, CHUNK), :],
                store_sems.at[s],
            )
            st.start()
            stores[i] = st

            if i >= 2:
                xdescs[i - 2].wait_send()
            out_bf16[s] = out_v[s].astype(jnp.bfloat16)
            xdescs[i] = x_rdma(i)
            xdescs[i].start()

            if i + 2 < N_CHUNKS:
                loads[i + 2] = start_loads(i + 2)

            if i >= 1:
                handle_xrecv(i - 1)

        handle_xrecv(N_CHUNKS - 1)
        stores[N_CHUNKS - 2].wait()
        stores[N_CHUNKS - 1].wait()
        xstores[N_CHUNKS - 1].wait()
        ydescs[N_CHUNKS - 2].wait_send()
        ydescs[N_CHUNKS - 1].wait_send()
        xdescs[N_CHUNKS - 2].wait_send()
        xdescs[N_CHUNKS - 1].wait_send()

    return pl.pallas_call(
        body,
        out_shape=jax.ShapeDtypeStruct((N_ROWS, N_COLS), jnp.float32),
        in_specs=[
            pl.BlockSpec(memory_space=pl.ANY),
            pl.BlockSpec(memory_space=pl.ANY),
            pl.BlockSpec(memory_space=pltpu.VMEM),
        ],
        out_specs=pl.BlockSpec(memory_space=pl.ANY),
        scratch_shapes=[
            pltpu.VMEM((2, CHUNK, N_COLS), jnp.float32),
            pltpu.VMEM((2, CHUNK, N_COLS), jnp.float32),
            pltpu.VMEM((2, CHUNK, N_COLS), jnp.bfloat16),
            pltpu.VMEM((2, CHUNK, N_COLS), jnp.bfloat16),
            pltpu.VMEM((2, CHUNK, N_COLS), jnp.float32),
            pltpu.VMEM((2, CHUNK, N_COLS), jnp.bfloat16),
            pltpu.VMEM((4, CHUNK, N_COLS), jnp.bfloat16),
            pltpu.VMEM((CHUNK, N_COLS), jnp.float32),
            pltpu.SemaphoreType.DMA((2,)),
            pltpu.SemaphoreType.DMA((2,)),
            pltpu.SemaphoreType.DMA((2,)),
            pltpu.SemaphoreType.DMA,
            pltpu.SemaphoreType.DMA((2,)),
            pltpu.SemaphoreType.DMA((2,)),
            pltpu.SemaphoreType.DMA((2,)),
            pltpu.SemaphoreType.DMA((4,)),
        ],
        compiler_params=pltpu.CompilerParams(
            collective_id=0,
            vmem_limit_bytes=60 * 1024 * 1024,
        ),
    )(partial, resid, gamma2)


# device time: 259148 ns/iter; 2.0919x vs baseline; 1.0519x over previous
import jax
import jax.numpy as jnp
from jax import lax
from jax.experimental import pallas as pl
from jax.experimental.pallas import tpu as pltpu

N_ROWS = 8192
N_COLS = 2048
HALF = N_ROWS // 2
CHUNK = 512
N_CHUNKS = HALF // CHUNK
EPS = 1e-6


def kernel(partial, resid, gamma):
    gamma2 = gamma.reshape(1, N_COLS)

    def body(
        partial_ref,
        resid_ref,
        gamma_ref,
        out_ref,
        own_f32,
        resid_v,
        own_bf16,
        recv_y,
        out_v,
        out_bf16,
        recv_x,
        xrecv_f32,
        pload_sems,
        rload_sems,
        store_sems,
        xstore_sem,
        ysend_sems,
        yrecv_sems,
        xsend_sems,
        xrecv_sems,
    ):
        my_x = lax.axis_index("x")
        my_y = lax.axis_index("y")
        ynbr = (my_x, 1 - my_y)
        xnbr = (1 - my_x, my_y)
        base_own = my_x * HALF
        base_oth = (1 - my_x) * HALF

        barrier_sem = pltpu.get_barrier_semaphore()
        for nbr in (ynbr, xnbr):
            pl.semaphore_signal(
                barrier_sem, inc=1, device_id=nbr,
                device_id_type=pl.DeviceIdType.MESH,
            )
        pl.semaphore_wait(barrier_sem, 2)

        def start_loads(i):
            s = i % 2
            rows = pl.ds(base_own + i * CHUNK, CHUNK)
            ld_p = pltpu.make_async_copy(
                partial_ref.at[0, rows, :], own_f32.at[s], pload_sems.at[s]
            )
            ld_r = pltpu.make_async_copy(
                resid_ref.at[rows, :], resid_v.at[s], rload_sems.at[s]
            )
            ld_p.start()
            ld_r.start()
            return ld_p, ld_r

        def y_rdma(i):
            return pltpu.make_async_remote_copy(
                src_ref=own_bf16.at[i % 2],
                dst_ref=recv_y.at[i % 4],
                send_sem=ysend_sems.at[i % 2],
                recv_sem=yrecv_sems.at[i % 4],
                device_id=ynbr,
                device_id_type=pl.DeviceIdType.MESH,
            )

        def x_rdma(i):
            return pltpu.make_async_remote_copy(
                src_ref=out_bf16.at[i % 2],
                dst_ref=recv_x.at[i % 4],
                send_sem=xsend_sems.at[i % 2],
                recv_sem=xrecv_sems.at[i % 4],
                device_id=xnbr,
                device_id_type=pl.DeviceIdType.MESH,
            )

        loads = [None] * N_CHUNKS
        ydescs = [None] * N_CHUNKS
        xdescs = [None] * N_CHUNKS
        stores = [None] * N_CHUNKS
        xstores = [None] * N_CHUNKS

        def handle_xrecv(j):
            xdescs[j].wait_recv()
            if j >= 1:
                xstores[j - 1].wait()
            xrecv_f32[...] = recv_x[j % 4].astype(jnp.float32)
            st = pltpu.make_async_copy(
                xrecv_f32,
                out_ref.at[pl.ds(base_oth + j * CHUNK, CHUNK), :],
                xstore_sem,
            )
            st.start()
            xstores[j] = st

        loads[0] = start_loads(0)
        loads[1] = start_loads(1)
        loads[0][0].wait()
        own_bf16[0] = own_f32[0].astype(jnp.bfloat16)
        ydescs[0] = y_rdma(0)
        ydescs[0].start()

        for i in range(N_CHUNKS):
            s = i % 2
            if i + 1 < N_CHUNKS:
                loads[i + 1][0].wait()
                if i >= 1:
                    ydescs[i - 1].wait_send()
                own_bf16[(i + 1) % 2] = own_f32[(i + 1) % 2].astype(jnp.bfloat16)
                ydescs[i + 1] = y_rdma(i + 1)
                ydescs[i + 1].start()

            ydescs[i].wait_recv()

            loads[i][1].wait()
            if i >= 2:
                stores[i - 2].wait()
            y = own_f32[s] + recv_y[i % 4].astype(jnp.float32) + resid_v[s]
            ms = jnp.mean(y * y, axis=-1, keepdims=True)
            out_v[s] = y * lax.rsqrt(ms + EPS) * gamma_ref[...]

            st = pltpu.make_async_copy(
                out_v.at[s],
                out_ref.at[pl.ds(base_own + i * CHUNK, CHUNK), :],
                store_sems.at[s],
            )
            st.start()
            stores[i] = st

            if i >= 2:
                xdescs[i - 2].wait_send()
            out_bf16[s] = out_v[s].astype(jnp.bfloat16)
            xdescs[i] = x_rdma(i)
            xdescs[i].start()

            if i + 2 < N_CHUNKS:
                loads[i + 2] = start_loads(i + 2)

            if i >= 1:
                handle_xrecv(i - 1)

        handle_xrecv(N_CHUNKS - 1)
        stores[N_CHUNKS - 2].wait()
        stores[N_CHUNKS - 1].wait()
        xstores[N_CHUNKS - 1].wait()
        ydescs[N_CHUNKS - 2].wait_send()
        ydescs[N_CHUNKS - 1].wait_send()
        xdescs[N_CHUNKS - 2].wait_send()
        xdescs[N_CHUNKS - 1].wait_send()

    return pl.pallas_call(
        body,
        out_shape=jax.ShapeDtypeStruct((N_ROWS, N_COLS), jnp.float32),
        in_specs=[
            pl.BlockSpec(memory_space=pl.ANY),
            pl.BlockSpec(memory_space=pl.ANY),
            pl.BlockSpec(memory_space=pltpu.VMEM),
        ],
        out_specs=pl.BlockSpec(memory_space=pl.ANY),
        scratch_shapes=[
            pltpu.VMEM((2, CHUNK, N_COLS), jnp.float32),
            pltpu.VMEM((2, CHUNK, N_COLS), jnp.float32),
            pltpu.VMEM((2, CHUNK, N_COLS), jnp.bfloat16),
            pltpu.VMEM((4, CHUNK, N_COLS), jnp.bfloat16),
            pltpu.VMEM((2, CHUNK, N_COLS), jnp.float32),
            pltpu.VMEM((2, CHUNK, N_COLS), jnp.bfloat16),
            pltpu.VMEM((4, CHUNK, N_COLS), jnp.bfloat16),
            pltpu.VMEM((CHUNK, N_COLS), jnp.float32),
            pltpu.SemaphoreType.DMA((2,)),
            pltpu.SemaphoreType.DMA((2,)),
            pltpu.SemaphoreType.DMA((2,)),
            pltpu.SemaphoreType.DMA,
            pltpu.SemaphoreType.DMA((2,)),
            pltpu.SemaphoreType.DMA((4,)),
            pltpu.SemaphoreType.DMA((2,)),
            pltpu.SemaphoreType.DMA((4,)),
        ],
        compiler_params=pltpu.CompilerParams(
            collective_id=0,
            vmem_limit_bytes=60 * 1024 * 1024,
        ),
    )(partial, resid, gamma2)


# device time: 236086 ns/iter; 2.2962x vs baseline; 1.0977x over previous
import jax
import jax.numpy as jnp
from jax import lax
from jax.experimental import pallas as pl
from jax.experimental.pallas import tpu as pltpu

N_ROWS = 8192
N_COLS = 2048
HALF = N_ROWS // 2
CHUNK = 512
N_CHUNKS = HALF // CHUNK
EPS = 1e-6


def kernel(partial, resid, gamma):
    gamma2 = gamma.reshape(1, N_COLS)

    def body(
        partial_ref,
        resid_ref,
        gamma_ref,
        out_ref,
        own_f32,
        resid_v,
        own_bf16,
        recv_y,
        out_v,
        out_bf16,
        recv_x,
        pload_sems,
        rload_sems,
        store_sems,
        xstore_sems,
        ysend_sems,
        yrecv_sems,
        xsend_sems,
        xrecv_sems,
    ):
        my_x = lax.axis_index("x")
        my_y = lax.axis_index("y")
        ynbr = (my_x, 1 - my_y)
        xnbr = (1 - my_x, my_y)
        base_own = my_x * HALF
        base_oth = (1 - my_x) * HALF

        barrier_sem = pltpu.get_barrier_semaphore()
        for nbr in (ynbr, xnbr):
            pl.semaphore_signal(
                barrier_sem, inc=1, device_id=nbr,
                device_id_type=pl.DeviceIdType.MESH,
            )
        pl.semaphore_wait(barrier_sem, 2)

        def start_loads(i):
            s = i % 2
            rows = pl.ds(base_own + i * CHUNK, CHUNK)
            ld_p = pltpu.make_async_copy(
                partial_ref.at[0, rows, :], own_f32.at[s], pload_sems.at[s]
            )
            ld_r = pltpu.make_async_copy(
                resid_ref.at[rows, :], resid_v.at[s], rload_sems.at[s]
            )
            ld_p.start()
            ld_r.start()
            return ld_p, ld_r

        def y_rdma(i):
            return pltpu.make_async_remote_copy(
                src_ref=own_bf16.at[i % 2],
                dst_ref=recv_y.at[i % 4],
                send_sem=ysend_sems.at[i % 2],
                recv_sem=yrecv_sems.at[i % 4],
                device_id=ynbr,
                device_id_type=pl.DeviceIdType.MESH,
            )

        def x_rdma(i):
            return pltpu.make_async_remote_copy(
                src_ref=out_bf16.at[i % 2],
                dst_ref=recv_x.at[i % 4],
                send_sem=xsend_sems.at[i % 2],
                recv_sem=xrecv_sems.at[i % 4],
                device_id=xnbr,
                device_id_type=pl.DeviceIdType.MESH,
            )

        loads = [None] * N_CHUNKS
        ydescs = [None] * N_CHUNKS
        xdescs = [None] * N_CHUNKS
        stores = [None] * N_CHUNKS
        xstores = [None] * N_CHUNKS

        def handle_xrecv(j):
            xdescs[j].wait_recv()
            st = pltpu.make_async_copy(
                recv_x.at[j % 4],
                out_ref.at[pl.ds(base_oth + j * CHUNK, CHUNK), :],
                xstore_sems.at[j % 2],
            )
            st.start()
            xstores[j] = st

        loads[0] = start_loads(0)
        loads[1] = start_loads(1)
        loads[0][0].wait()
        own_bf16[0] = own_f32[0].astype(jnp.bfloat16)
        ydescs[0] = y_rdma(0)
        ydescs[0].start()

        for i in range(N_CHUNKS):
            s = i % 2
            if i + 1 < N_CHUNKS:
                loads[i + 1][0].wait()
                if i >= 1:
                    ydescs[i - 1].wait_send()
                own_bf16[(i + 1) % 2] = own_f32[(i + 1) % 2].astype(jnp.bfloat16)
                ydescs[i + 1] = y_rdma(i + 1)
                ydescs[i + 1].start()

            ydescs[i].wait_recv()

            loads[i][1].wait()
            y = own_f32[s] + recv_y[i % 4].astype(jnp.float32) + resid_v[s]
            ms = jnp.mean(y * y, axis=-1, keepdims=True)
            out_v[...] = y * lax.rsqrt(ms + EPS) * gamma_ref[...]

            if i >= 2:
                xdescs[i - 2].wait_send()
                stores[i - 2].wait()
                xstores[i - 2].wait()
            out_bf16[s] = out_v[...].astype(jnp.bfloat16)
            st = pltpu.make_async_copy(
                out_bf16.at[s],
                out_ref.at[pl.ds(base_own + i * CHUNK, CHUNK), :],
                store_sems.at[s],
            )
            st.start()
            stores[i] = st
            xdescs[i] = x_rdma(i)
            xdescs[i].start()

            if i + 2 < N_CHUNKS:
                loads[i + 2] = start_loads(i + 2)

            if i >= 1:
                handle_xrecv(i - 1)

        handle_xrecv(N_CHUNKS - 1)
        for k in (N_CHUNKS - 2, N_CHUNKS - 1):
            stores[k].wait()
            xstores[k].wait()
            ydescs[k].wait_send()
            xdescs[k].wait_send()

    return pl.pallas_call(
        body,
        out_shape=jax.ShapeDtypeStruct((N_ROWS, N_COLS), jnp.bfloat16),
        in_specs=[
            pl.BlockSpec(memory_space=pl.ANY),
            pl.BlockSpec(memory_space=pl.ANY),
            pl.BlockSpec(memory_space=pltpu.VMEM),
        ],
        out_specs=pl.BlockSpec(memory_space=pl.ANY),
        scratch_shapes=[
            pltpu.VMEM((2, CHUNK, N_COLS), jnp.float32),
            pltpu.VMEM((2, CHUNK, N_COLS), jnp.float32),
            pltpu.VMEM((2, CHUNK, N_COLS), jnp.bfloat16),
            pltpu.VMEM((4, CHUNK, N_COLS), jnp.bfloat16),
            pltpu.VMEM((CHUNK, N_COLS), jnp.float32),
            pltpu.VMEM((2, CHUNK, N_COLS), jnp.bfloat16),
            pltpu.VMEM((4, CHUNK, N_COLS), jnp.bfloat16),
            pltpu.SemaphoreType.DMA((2,)),
            pltpu.SemaphoreType.DMA((2,)),
            pltpu.SemaphoreType.DMA((2,)),
            pltpu.SemaphoreType.DMA((2,)),
            pltpu.SemaphoreType.DMA((2,)),
            pltpu.SemaphoreType.DMA((4,)),
            pltpu.SemaphoreType.DMA((2,)),
            pltpu.SemaphoreType.DMA((4,)),
        ],
        compiler_params=pltpu.CompilerParams(
            collective_id=0,
            vmem_limit_bytes=60 * 1024 * 1024,
        ),
    )(partial, resid, gamma2)


# device time: 223046 ns/iter; 2.4305x vs baseline; 1.0585x over previous
import jax
import jax.numpy as jnp
from jax import lax
from jax.experimental import pallas as pl
from jax.experimental.pallas import tpu as pltpu

N_ROWS = 8192
N_COLS = 2048
HALF = N_ROWS // 2
CHUNK = 256
N_CHUNKS = HALF // CHUNK
EPS = 1e-6


def kernel(partial, resid, gamma):
    gamma2 = gamma.reshape(1, N_COLS)

    def body(
        partial_ref,
        resid_ref,
        gamma_ref,
        out_ref,
        own_f32,
        resid_v,
        own_bf16,
        recv_y,
        out_v,
        out_bf16,
        recv_x,
        pload_sems,
        rload_sems,
        store_sems,
        xstore_sems,
        ysend_sems,
        yrecv_sems,
        xsend_sems,
        xrecv_sems,
    ):
        my_x = lax.axis_index("x")
        my_y = lax.axis_index("y")
        ynbr = (my_x, 1 - my_y)
        xnbr = (1 - my_x, my_y)
        base_own = my_x * HALF
        base_oth = (1 - my_x) * HALF

        barrier_sem = pltpu.get_barrier_semaphore()
        for nbr in (ynbr, xnbr):
            pl.semaphore_signal(
                barrier_sem, inc=1, device_id=nbr,
                device_id_type=pl.DeviceIdType.MESH,
            )
        pl.semaphore_wait(barrier_sem, 2)

        def start_loads(i):
            s = i % 2
            rows = pl.ds(base_own + i * CHUNK, CHUNK)
            ld_p = pltpu.make_async_copy(
                partial_ref.at[0, rows, :], own_f32.at[s], pload_sems.at[s]
            )
            ld_r = pltpu.make_async_copy(
                resid_ref.at[rows, :], resid_v.at[s], rload_sems.at[s]
            )
            ld_p.start()
            ld_r.start()
            return ld_p, ld_r

        def y_rdma(i):
            return pltpu.make_async_remote_copy(
                src_ref=own_bf16.at[i % 2],
                dst_ref=recv_y.at[i % 4],
                send_sem=ysend_sems.at[i % 2],
                recv_sem=yrecv_sems.at[i % 4],
                device_id=ynbr,
                device_id_type=pl.DeviceIdType.MESH,
            )

        def x_rdma(i):
            return pltpu.make_async_remote_copy(
                src_ref=out_bf16.at[i % 2],
                dst_ref=recv_x.at[i % 4],
                send_sem=xsend_sems.at[i % 2],
                recv_sem=xrecv_sems.at[i % 4],
                device_id=xnbr,
                device_id_type=pl.DeviceIdType.MESH,
            )

        loads = [None] * N_CHUNKS
        ydescs = [None] * N_CHUNKS
        xdescs = [None] * N_CHUNKS
        stores = [None] * N_CHUNKS
        xstores = [None] * N_CHUNKS

        def handle_xrecv(j):
            xdescs[j].wait_recv()
            st = pltpu.make_async_copy(
                recv_x.at[j % 4],
                out_ref.at[pl.ds(base_oth + j * CHUNK, CHUNK), :],
                xstore_sems.at[j % 2],
            )
            st.start()
            xstores[j] = st

        loads[0] = start_loads(0)
        loads[1] = start_loads(1)
        loads[0][0].wait()
        own_bf16[0] = own_f32[0].astype(jnp.bfloat16)
        ydescs[0] = y_rdma(0)
        ydescs[0].start()

        for i in range(N_CHUNKS):
            s = i % 2
            if i + 1 < N_CHUNKS:
                loads[i + 1][0].wait()
                if i >= 1:
                    ydescs[i - 1].wait_send()
                own_bf16[(i + 1) % 2] = own_f32[(i + 1) % 2].astype(jnp.bfloat16)
                ydescs[i + 1] = y_rdma(i + 1)
                ydescs[i + 1].start()

            ydescs[i].wait_recv()

            loads[i][1].wait()
            y = own_f32[s] + recv_y[i % 4].astype(jnp.float32) + resid_v[s]
            ms = jnp.mean(y * y, axis=-1, keepdims=True)
            out_v[...] = y * lax.rsqrt(ms + EPS) * gamma_ref[...]

            if i >= 2:
                xdescs[i - 2].wait_send()
                stores[i - 2].wait()
                xstores[i - 2].wait()
            out_bf16[s] = out_v[...].astype(jnp.bfloat16)
            st = pltpu.make_async_copy(
                out_bf16.at[s],
                out_ref.at[pl.ds(base_own + i * CHUNK, CHUNK), :],
                store_sems.at[s],
            )
            st.start()
            stores[i] = st
            xdescs[i] = x_rdma(i)
            xdescs[i].start()

            if i + 2 < N_CHUNKS:
                loads[i + 2] = start_loads(i + 2)

            if i >= 1:
                handle_xrecv(i - 1)

        handle_xrecv(N_CHUNKS - 1)
        for k in (N_CHUNKS - 2, N_CHUNKS - 1):
            stores[k].wait()
            xstores[k].wait()
            ydescs[k].wait_send()
            xdescs[k].wait_send()

    return pl.pallas_call(
        body,
        out_shape=jax.ShapeDtypeStruct((N_ROWS, N_COLS), jnp.bfloat16),
        in_specs=[
            pl.BlockSpec(memory_space=pl.ANY),
            pl.BlockSpec(memory_space=pl.ANY),
            pl.BlockSpec(memory_space=pltpu.VMEM),
        ],
        out_specs=pl.BlockSpec(memory_space=pl.ANY),
        scratch_shapes=[
            pltpu.VMEM((2, CHUNK, N_COLS), jnp.float32),
            pltpu.VMEM((2, CHUNK, N_COLS), jnp.float32),
            pltpu.VMEM((2, CHUNK, N_COLS), jnp.bfloat16),
            pltpu.VMEM((4, CHUNK, N_COLS), jnp.bfloat16),
            pltpu.VMEM((CHUNK, N_COLS), jnp.float32),
            pltpu.VMEM((2, CHUNK, N_COLS), jnp.bfloat16),
            pltpu.VMEM((4, CHUNK, N_COLS), jnp.bfloat16),
            pltpu.SemaphoreType.DMA((2,)),
            pltpu.SemaphoreType.DMA((2,)),
            pltpu.SemaphoreType.DMA((2,)),
            pltpu.SemaphoreType.DMA((2,)),
            pltpu.SemaphoreType.DMA((2,)),
            pltpu.SemaphoreType.DMA((4,)),
            pltpu.SemaphoreType.DMA((2,)),
            pltpu.SemaphoreType.DMA((4,)),
        ],
        compiler_params=pltpu.CompilerParams(
            collective_id=0,
            vmem_limit_bytes=60 * 1024 * 1024,
        ),
    )(partial, resid, gamma2)


# device time: 216408 ns/iter; 2.5050x vs baseline; 1.0307x over previous
import jax
import jax.numpy as jnp
from jax import lax
from jax.experimental import pallas as pl
from jax.experimental.pallas import tpu as pltpu

N_ROWS = 8192
N_COLS = 2048
HALF = N_ROWS // 2
CHUNK = 128
N_CHUNKS = HALF // CHUNK
EPS = 1e-6


def kernel(partial, resid, gamma):
    gamma2 = gamma.reshape(1, N_COLS)

    def body(
        partial_ref,
        resid_ref,
        gamma_ref,
        out_ref,
        own_f32,
        resid_v,
        own_bf16,
        recv_y,
        out_v,
        out_bf16,
        recv_x,
        pload_sems,
        rload_sems,
        store_sems,
        xstore_sems,
        ysend_sems,
        yrecv_sems,
        xsend_sems,
        xrecv_sems,
    ):
        my_x = lax.axis_index("x")
        my_y = lax.axis_index("y")
        ynbr = (my_x, 1 - my_y)
        xnbr = (1 - my_x, my_y)
        base_own = my_x * HALF
        base_oth = (1 - my_x) * HALF

        barrier_sem = pltpu.get_barrier_semaphore()
        for nbr in (ynbr, xnbr):
            pl.semaphore_signal(
                barrier_sem, inc=1, device_id=nbr,
                device_id_type=pl.DeviceIdType.MESH,
            )
        pl.semaphore_wait(barrier_sem, 2)

        def start_loads(i):
            s = i % 2
            rows = pl.ds(base_own + i * CHUNK, CHUNK)
            ld_p = pltpu.make_async_copy(
                partial_ref.at[0, rows, :], own_f32.at[s], pload_sems.at[s]
            )
            ld_r = pltpu.make_async_copy(
                resid_ref.at[rows, :], resid_v.at[s], rload_sems.at[s]
            )
            ld_p.start()
            ld_r.start()
            return ld_p, ld_r

        def y_rdma(i):
            return pltpu.make_async_remote_copy(
                src_ref=own_bf16.at[i % 2],
                dst_ref=recv_y.at[i % 4],
                send_sem=ysend_sems.at[i % 2],
                recv_sem=yrecv_sems.at[i % 4],
                device_id=ynbr,
                device_id_type=pl.DeviceIdType.MESH,
            )

        def x_rdma(i):
            return pltpu.make_async_remote_copy(
                src_ref=out_bf16.at[i % 2],
                dst_ref=recv_x.at[i % 4],
                send_sem=xsend_sems.at[i % 2],
                recv_sem=xrecv_sems.at[i % 4],
                device_id=xnbr,
                device_id_type=pl.DeviceIdType.MESH,
            )

        loads = [None] * N_CHUNKS
        ydescs = [None] * N_CHUNKS
        xdescs = [None] * N_CHUNKS
        stores = [None] * N_CHUNKS
        xstores = [None] * N_CHUNKS

        def handle_xrecv(j):
            xdescs[j].wait_recv()
            st = pltpu.make_async_copy(
                recv_x.at[j % 4],
                out_ref.at[pl.ds(base_oth + j * CHUNK, CHUNK), :],
                xstore_sems.at[j % 2],
            )
            st.start()
            xstores[j] = st

        loads[0] = start_loads(0)
        loads[1] = start_loads(1)
        loads[0][0].wait()
        own_bf16[0] = own_f32[0].astype(jnp.bfloat16)
        ydescs[0] = y_rdma(0)
        ydescs[0].start()

        for i in range(N_CHUNKS):
            s = i % 2
            if i + 1 < N_CHUNKS:
                loads[i + 1][0].wait()
                if i >= 1:
                    ydescs[i - 1].wait_send()
                own_bf16[(i + 1) % 2] = own_f32[(i + 1) % 2].astype(jnp.bfloat16)
                ydescs[i + 1] = y_rdma(i + 1)
                ydescs[i + 1].start()

            ydescs[i].wait_recv()

            loads[i][1].wait()
            y = own_f32[s] + recv_y[i % 4].astype(jnp.float32) + resid_v[s]
            ms = jnp.mean(y * y, axis=-1, keepdims=True)
            out_v[...] = y * lax.rsqrt(ms + EPS) * gamma_ref[...]

            if i >= 2:
                xdescs[i - 2].wait_send()
                stores[i - 2].wait()
                xstores[i - 2].wait()
            out_bf16[s] = out_v[...].astype(jnp.bfloat16)
            st = pltpu.make_async_copy(
                out_bf16.at[s],
                out_ref.at[pl.ds(base_own + i * CHUNK, CHUNK), :],
                store_sems.at[s],
            )
            st.start()
            stores[i] = st
            xdescs[i] = x_rdma(i)
            xdescs[i].start()

            if i + 2 < N_CHUNKS:
                loads[i + 2] = start_loads(i + 2)

            if i >= 1:
                handle_xrecv(i - 1)

        handle_xrecv(N_CHUNKS - 1)
        for k in (N_CHUNKS - 2, N_CHUNKS - 1):
            stores[k].wait()
            xstores[k].wait()
            ydescs[k].wait_send()
            xdescs[k].wait_send()

    return pl.pallas_call(
        body,
        out_shape=jax.ShapeDtypeStruct((N_ROWS, N_COLS), jnp.bfloat16),
        in_specs=[
            pl.BlockSpec(memory_space=pl.ANY),
            pl.BlockSpec(memory_space=pl.ANY),
            pl.BlockSpec(memory_space=pltpu.VMEM),
        ],
        out_specs=pl.BlockSpec(memory_space=pl.ANY),
        scratch_shapes=[
            pltpu.VMEM((2, CHUNK, N_COLS), jnp.float32),
            pltpu.VMEM((2, CHUNK, N_COLS), jnp.float32),
            pltpu.VMEM((2, CHUNK, N_COLS), jnp.bfloat16),
            pltpu.VMEM((4, CHUNK, N_COLS), jnp.bfloat16),
            pltpu.VMEM((CHUNK, N_COLS), jnp.float32),
            pltpu.VMEM((2, CHUNK, N_COLS), jnp.bfloat16),
            pltpu.VMEM((4, CHUNK, N_COLS), jnp.bfloat16),
            pltpu.SemaphoreType.DMA((2,)),
            pltpu.SemaphoreType.DMA((2,)),
            pltpu.SemaphoreType.DMA((2,)),
            pltpu.SemaphoreType.DMA((2,)),
            pltpu.SemaphoreType.DMA((2,)),
            pltpu.SemaphoreType.DMA((4,)),
            pltpu.SemaphoreType.DMA((2,)),
            pltpu.SemaphoreType.DMA((4,)),
        ],
        compiler_params=pltpu.CompilerParams(
            collective_id=0,
            vmem_limit_bytes=60 * 1024 * 1024,
        ),
    )(partial, resid, gamma2)


# device time: 216346 ns/iter; 2.5058x vs baseline; 1.0003x over previous
import jax
import jax.numpy as jnp
from jax import lax
from jax.experimental import pallas as pl
from jax.experimental.pallas import tpu as pltpu

N_ROWS = 8192
N_COLS = 2048
HALF = N_ROWS // 2
CHUNK = 128
N_CHUNKS = HALF // CHUNK
EPS = 1e-6


def kernel(partial, resid, gamma):
    gamma2 = gamma.reshape(1, N_COLS)

    def body(
        partial_ref,
        resid_ref,
        gamma_ref,
        out_ref,
        own_f32,
        resid_v,
        own_bf16,
        recv_y,
        out_v,
        out_bf16,
        recv_x,
        pload_sems,
        rload_sems,
        store_sems,
        xstore_sems,
        ysend_sems,
        yrecv_sems,
        xsend_sems,
        xrecv_sems,
    ):
        my_x = lax.axis_index("x")
        my_y = lax.axis_index("y")
        ynbr = (my_x, 1 - my_y)
        xnbr = (1 - my_x, my_y)
        base_own = my_x * HALF
        base_oth = (1 - my_x) * HALF

        barrier_sem = pltpu.get_barrier_semaphore()
        for nbr in (ynbr, xnbr):
            pl.semaphore_signal(
                barrier_sem, inc=1, device_id=nbr,
                device_id_type=pl.DeviceIdType.MESH,
            )
        pl.semaphore_wait(barrier_sem, 2)

        def start_loads(i):
            s = i % 2
            rows = pl.ds(base_own + i * CHUNK, CHUNK)
            ld_p = pltpu.make_async_copy(
                partial_ref.at[0, rows, :], own_f32.at[s], pload_sems.at[s]
            )
            ld_r = pltpu.make_async_copy(
                resid_ref.at[rows, :], resid_v.at[s], rload_sems.at[s]
            )
            ld_p.start()
            ld_r.start()
            return ld_p, ld_r

        def y_rdma(i):
            return pltpu.make_async_remote_copy(
                src_ref=own_bf16.at[i % 2],
                dst_ref=recv_y.at[i % 4],
                send_sem=ysend_sems.at[i % 2],
                recv_sem=yrecv_sems.at[i % 4],
                device_id=ynbr,
                device_id_type=pl.DeviceIdType.MESH,
            )

        def x_rdma(i):
            return pltpu.make_async_remote_copy(
                src_ref=out_bf16.at[i % 2],
                dst_ref=recv_x.at[i % 4],
                send_sem=xsend_sems.at[i % 2],
                recv_sem=xrecv_sems.at[i % 4],
                device_id=xnbr,
                device_id_type=pl.DeviceIdType.MESH,
            )

        loads = [None] * N_CHUNKS
        ydescs = [None] * N_CHUNKS
        xdescs = [None] * N_CHUNKS
        stores = [None] * N_CHUNKS
        xstores = [None] * N_CHUNKS

        def handle_xrecv(j):
            xdescs[j].wait_recv()
            st = pltpu.make_async_copy(
                recv_x.at[j % 4],
                out_ref.at[pl.ds(base_oth + j * CHUNK, CHUNK), :],
                xstore_sems.at[j % 2],
            )
            st.start()
            xstores[j] = st

        loads[0] = start_loads(0)
        loads[1] = start_loads(1)
        loads[0][0].wait()
        own_bf16[0] = own_f32[0].astype(jnp.bfloat16)
        ydescs[0] = y_rdma(0)
        ydescs[0].start()

        for i in range(N_CHUNKS):
            s = i % 2
            if i + 1 < N_CHUNKS:
                loads[i + 1][0].wait()
                if i >= 1:
                    ydescs[i - 1].wait_send()
                own_bf16[(i + 1) % 2] = own_f32[(i + 1) % 2].astype(jnp.bfloat16)
                ydescs[i + 1] = y_rdma(i + 1)
                ydescs[i + 1].start()

            ydescs[i].wait_recv()

            loads[i][1].wait()
            y = own_f32[s] + recv_y[i % 4].astype(jnp.float32) + resid_v[s]
            ms = jnp.mean(y * y, axis=-1, keepdims=True)
            out_v[...] = y * lax.rsqrt(ms + EPS) * gamma_ref[...]

            if i >= 2:
                xdescs[i - 2].wait_send()
                stores[i - 2].wait()
                xstores[i - 2].wait()
            out_bf16[s] = out_v[...].astype(jnp.bfloat16)
            st = pltpu.make_async_copy(
                out_bf16.at[s],
                out_ref.at[pl.ds(base_own + i * CHUNK, CHUNK), :],
                store_sems.at[s],
            )
            st.start()
            stores[i] = st
            xdescs[i] = x_rdma(i)
            xdescs[i].start()

            if i + 2 < N_CHUNKS:
                loads[i + 2] = start_loads(i + 2)

            if i >= 1:
                handle_xrecv(i - 1)

        handle_xrecv(N_CHUNKS - 1)
        for k in (N_CHUNKS - 2, N_CHUNKS - 1):
            stores[k].wait()
            xstores[k].wait()
            ydescs[k].wait_send()
            xdescs[k].wait_send()

    return pl.pallas_call(
        body,
        out_shape=jax.ShapeDtypeStruct((N_ROWS, N_COLS), jnp.bfloat16),
        in_specs=[
            pl.BlockSpec(memory_space=pltpu.MemorySpace.HBM),
            pl.BlockSpec(memory_space=pltpu.MemorySpace.HBM),
            pl.BlockSpec(memory_space=pltpu.VMEM),
        ],
        out_specs=pl.BlockSpec(memory_space=pltpu.MemorySpace.HBM),
        scratch_shapes=[
            pltpu.VMEM((2, CHUNK, N_COLS), jnp.float32),
            pltpu.VMEM((2, CHUNK, N_COLS), jnp.float32),
            pltpu.VMEM((2, CHUNK, N_COLS), jnp.bfloat16),
            pltpu.VMEM((4, CHUNK, N_COLS), jnp.bfloat16),
            pltpu.VMEM((CHUNK, N_COLS), jnp.float32),
            pltpu.VMEM((2, CHUNK, N_COLS), jnp.bfloat16),
            pltpu.VMEM((4, CHUNK, N_COLS), jnp.bfloat16),
            pltpu.SemaphoreType.DMA((2,)),
            pltpu.SemaphoreType.DMA((2,)),
            pltpu.SemaphoreType.DMA((2,)),
            pltpu.SemaphoreType.DMA((2,)),
            pltpu.SemaphoreType.DMA((2,)),
            pltpu.SemaphoreType.DMA((4,)),
            pltpu.SemaphoreType.DMA((2,)),
            pltpu.SemaphoreType.DMA((4,)),
        ],
        compiler_params=pltpu.CompilerParams(
            collective_id=0,
            vmem_limit_bytes=60 * 1024 * 1024,
        ),
    )(partial, resid, gamma2)
